# baseline (speedup 1.0000x reference)
import functools

import jax
import jax.numpy as jnp
from jax import lax
from jax.experimental import pallas as pl
from jax.experimental.pallas import tpu as pltpu
from jax.experimental.pallas import tpu_sc as plsc

NC, NS, LANES = 2, 16, 16
NW = NC * NS
N_B, N_L, D_MODEL = 4096, 200, 128
BT = N_B // 128
LBLK = 10
NLBLK = N_L // LBLK
BG = 8


def _bcast(vec, j):
    return lax.gather(
        vec, jnp.full((LANES, 1), j, jnp.int32),
        lax.GatherDimensionNumbers(
            offset_dims=(), collapsed_slice_dims=(0,), start_index_map=(0,)),
        (1,), mode=lax.GatherScatterMode.PROMISE_IN_BOUNDS)


def _body(notes_h, w_h, ptab_h, vtab_h, out_h,
          nbuf, obuf0, obuf1, obuf2, obuf3, ptab_v, vtab_v, w_v,
          osem0, osem1, osem2, osem3):
    wid = lax.axis_index("s") * NC + lax.axis_index("c")

    pltpu.sync_copy(ptab_h, ptab_v)
    pltpu.sync_copy(vtab_h, vtab_v)
    pltpu.sync_copy(w_h, w_v)

    obufs = (obuf0, obuf1, obuf2, obuf3)
    osems = (osem0, osem1, osem2, osem3)

    iota = lax.iota(jnp.int32, LANES)
    iota16 = iota + LANES
    w0 = w_v[0, pl.ds(0, LANES)]
    w1 = w_v[0, pl.ds(LANES, LANES)]
    b0 = w_v[1, pl.ds(0, LANES)]
    b1 = w_v[1, pl.ds(LANES, LANES)]
    d0 = w_v[2, pl.ds(0, LANES)]
    d1 = w_v[2, pl.ds(LANES, LANES)]
    c0 = w_v[3, pl.ds(0, LANES)]
    c1 = w_v[3, pl.ds(LANES, LANES)]

    def out_slice(lblk, bg):
        return out_h.at[pl.ds(wid * 128 + bg * LANES, LANES),
                        pl.ds(lblk * LBLK, LBLK), :]

    def wait_out(b):
        pltpu.make_async_copy(obufs[b], out_slice(0, 0), osems[b]).wait()

    def compute(par, bg):
        obuf = obufs[par]
        j0 = bg * LANES

        @plsc.parallel_loop(0, LBLK)
        def seq_body(li):
            sv = nbuf[li, 0, pl.ds(j0, LANES)]
            dv = nbuf[li, 1, pl.ds(j0, LANES)]
            pv = nbuf[li, 2, pl.ds(j0, LANES)].astype(jnp.int32) * 32
            vv = nbuf[li, 3, pl.ds(j0, LANES)].astype(jnp.int32) * 32
            for j in range(LANES):
                s = _bcast(sv, j)
                d = _bcast(dv, j)
                pb = _bcast(pv, j)
                vb = _bcast(vv, j)
                p0 = plsc.load_gather(ptab_v, [pb + iota])
                p1 = plsc.load_gather(ptab_v, [pb + iota16])
                v0 = plsc.load_gather(vtab_v, [vb + iota])
                v1 = plsc.load_gather(vtab_v, [vb + iota16])
                obuf[j, li, pl.ds(0, LANES)] = s * w0 + b0
                obuf[j, li, pl.ds(16, LANES)] = s * w1 + b1
                obuf[j, li, pl.ds(32, LANES)] = d * d0 + c0
                obuf[j, li, pl.ds(48, LANES)] = d * d1 + c1
                obuf[j, li, pl.ds(64, LANES)] = p0
                obuf[j, li, pl.ds(80, LANES)] = p1
                obuf[j, li, pl.ds(96, LANES)] = v0
                obuf[j, li, pl.ds(112, LANES)] = v1

    def lblk_body(lblk, _):
        pltpu.sync_copy(notes_h.at[pl.ds(lblk * LBLK, LBLK), wid], nbuf)

        def pair_body(i, _):
            for par in range(4):
                bg = i * 4 + par
                step = lblk * BG + bg

                @pl.when(step >= 4)
                def _():
                    wait_out(par)

                compute(par, bg)
                pltpu.async_copy(obufs[par], out_slice(lblk, bg), osems[par])
            return 0

        lax.fori_loop(0, BG // 4, pair_body, 0)
        return 0

    lax.fori_loop(0, NLBLK, lblk_body, 0)
    for par in range(4):
        wait_out(par)


def kernel(notes, start_W, start_b, dur_W, dur_b, pitch_table, vel_table):
    notes_r = notes.transpose(1, 2, 0).reshape(N_L, 4, BT, 128).transpose(0, 2, 1, 3)
    w = jnp.stack([
        start_W.reshape(32), start_b.reshape(32),
        dur_W.reshape(32), dur_b.reshape(32),
    ])
    mesh = plsc.VectorSubcoreMesh(
        core_axis_name="c", subcore_axis_name="s",
        num_cores=NC, num_subcores=NS)
    run = functools.partial(
        pl.kernel,
        out_type=jax.ShapeDtypeStruct((N_B, N_L, D_MODEL), jnp.float32),
        mesh=mesh,
        compiler_params=pltpu.CompilerParams(
            needs_layout_passes=False, use_tc_tiling_on_sc=False),
        scratch_types=[
            pltpu.VMEM((LBLK, 4, 128), jnp.float32),
            pltpu.VMEM((LANES, LBLK, D_MODEL), jnp.float32),
            pltpu.VMEM((LANES, LBLK, D_MODEL), jnp.float32),
            pltpu.VMEM((LANES, LBLK, D_MODEL), jnp.float32),
            pltpu.VMEM((LANES, LBLK, D_MODEL), jnp.float32),
            pltpu.VMEM((88 * 32,), jnp.float32),
            pltpu.VMEM((17 * 32,), jnp.float32),
            pltpu.VMEM((4, 32), jnp.float32),
            pltpu.SemaphoreType.DMA,
            pltpu.SemaphoreType.DMA,
            pltpu.SemaphoreType.DMA,
            pltpu.SemaphoreType.DMA,
        ],
    )(_body)
    return run(notes_r, w, pitch_table.reshape(88 * 32), vel_table.reshape(17 * 32))


# speedup vs baseline: 1.3839x; 1.3839x over previous
import functools

import jax
import jax.numpy as jnp
from jax import lax
from jax.experimental import pallas as pl
from jax.experimental.pallas import tpu as pltpu
from jax.experimental.pallas import tpu_sc as plsc

NC, NS, LANES = 2, 16, 16
NW = NC * NS
N_B, N_L, D_MODEL = 4096, 200, 128
BT = N_B // 128
LBLK = 20
NLBLK = N_L // LBLK
BG = 8


def _bcast(vec, j):
    return lax.gather(
        vec, jnp.full((LANES, 1), j, jnp.int32),
        lax.GatherDimensionNumbers(
            offset_dims=(), collapsed_slice_dims=(0,), start_index_map=(0,)),
        (1,), mode=lax.GatherScatterMode.PROMISE_IN_BOUNDS)


def _body(notes_h, w_h, ptab_h, vtab_h, out_h,
          nbuf0, nbuf1, obuf0, obuf1, ptab_v, vtab_v, w_v,
          isem0, isem1, osem0, osem1):
    wid = lax.axis_index("s") * NC + lax.axis_index("c")

    pltpu.sync_copy(ptab_h, ptab_v)
    pltpu.sync_copy(vtab_h, vtab_v)
    pltpu.sync_copy(w_h, w_v)

    obufs = (obuf0, obuf1)
    osems = (osem0, osem1)
    nbufs = (nbuf0, nbuf1)
    isems = (isem0, isem1)

    def in_slice(lblk):
        return notes_h.at[pl.ds(lblk * LBLK, LBLK), wid]

    def start_in(lblk, b):
        pltpu.async_copy(in_slice(lblk), nbufs[b], isems[b])

    def wait_in(b):
        pltpu.make_async_copy(in_slice(0), nbufs[b], isems[b]).wait()

    iota = lax.iota(jnp.int32, LANES)
    iota16 = iota + LANES
    w0 = w_v[0, pl.ds(0, LANES)]
    w1 = w_v[0, pl.ds(LANES, LANES)]
    b0 = w_v[1, pl.ds(0, LANES)]
    b1 = w_v[1, pl.ds(LANES, LANES)]
    d0 = w_v[2, pl.ds(0, LANES)]
    d1 = w_v[2, pl.ds(LANES, LANES)]
    c0 = w_v[3, pl.ds(0, LANES)]
    c1 = w_v[3, pl.ds(LANES, LANES)]

    def out_slice(lblk, bg):
        return out_h.at[pl.ds(wid * 128 + bg * LANES, LANES),
                        pl.ds(lblk * LBLK, LBLK), :]

    def wait_out(b):
        pltpu.make_async_copy(obufs[b], out_slice(0, 0), osems[b]).wait()

    def compute(npar, par, bg):
        nbuf = nbufs[npar]
        obuf = obufs[par]
        j0 = bg * LANES

        @plsc.parallel_loop(0, LBLK)
        def seq_body(li):
            sv = nbuf[li, 0, pl.ds(j0, LANES)]
            dv = nbuf[li, 1, pl.ds(j0, LANES)]
            pv = nbuf[li, 2, pl.ds(j0, LANES)].astype(jnp.int32) * 32
            vv = nbuf[li, 3, pl.ds(j0, LANES)].astype(jnp.int32) * 32
            for j in range(LANES):
                s = _bcast(sv, j)
                d = _bcast(dv, j)
                pb = _bcast(pv, j)
                vb = _bcast(vv, j)
                p0 = plsc.load_gather(ptab_v, [pb + iota])
                p1 = plsc.load_gather(ptab_v, [pb + iota16])
                v0 = plsc.load_gather(vtab_v, [vb + iota])
                v1 = plsc.load_gather(vtab_v, [vb + iota16])
                obuf[j, li, pl.ds(0, LANES)] = s * w0 + b0
                obuf[j, li, pl.ds(16, LANES)] = s * w1 + b1
                obuf[j, li, pl.ds(32, LANES)] = d * d0 + c0
                obuf[j, li, pl.ds(48, LANES)] = d * d1 + c1
                obuf[j, li, pl.ds(64, LANES)] = p0
                obuf[j, li, pl.ds(80, LANES)] = p1
                obuf[j, li, pl.ds(96, LANES)] = v0
                obuf[j, li, pl.ds(112, LANES)] = v1

    start_in(0, 0)

    def lblk_pair_body(h, _):
        for npar in range(2):
            lblk = h * 2 + npar
            wait_in(npar)

            @pl.when(lblk + 1 < NLBLK)
            def _():
                start_in(lblk + 1, 1 - npar)

            def pair_body(i, _):
                for par in range(2):
                    bg = i * 2 + par
                    step = lblk * BG + bg

                    @pl.when(step >= 2)
                    def _():
                        wait_out(par)

                    compute(npar, par, bg)
                    pltpu.async_copy(obufs[par], out_slice(lblk, bg),
                                     osems[par])
                return 0

            lax.fori_loop(0, BG // 2, pair_body, 0)
        return 0

    lax.fori_loop(0, NLBLK // 2, lblk_pair_body, 0)
    wait_out(0)
    wait_out(1)


def kernel(notes, start_W, start_b, dur_W, dur_b, pitch_table, vel_table):
    notes_r = notes.transpose(1, 2, 0).reshape(N_L, 4, BT, 128).transpose(0, 2, 1, 3)
    w = jnp.stack([
        start_W.reshape(32), start_b.reshape(32),
        dur_W.reshape(32), dur_b.reshape(32),
    ])
    mesh = plsc.VectorSubcoreMesh(
        core_axis_name="c", subcore_axis_name="s",
        num_cores=NC, num_subcores=NS)
    run = functools.partial(
        pl.kernel,
        out_type=jax.ShapeDtypeStruct((N_B, N_L, D_MODEL), jnp.float32),
        mesh=mesh,
        compiler_params=pltpu.CompilerParams(
            needs_layout_passes=False, use_tc_tiling_on_sc=False),
        scratch_types=[
            pltpu.VMEM((LBLK, 4, 128), jnp.float32),
            pltpu.VMEM((LBLK, 4, 128), jnp.float32),
            pltpu.VMEM((LANES, LBLK, D_MODEL), jnp.float32),
            pltpu.VMEM((LANES, LBLK, D_MODEL), jnp.float32),
            pltpu.VMEM((88 * 32,), jnp.float32),
            pltpu.VMEM((17 * 32,), jnp.float32),
            pltpu.VMEM((4, 32), jnp.float32),
            pltpu.SemaphoreType.DMA,
            pltpu.SemaphoreType.DMA,
            pltpu.SemaphoreType.DMA,
            pltpu.SemaphoreType.DMA,
        ],
    )(_body)
    return run(notes_r, w, pitch_table.reshape(88 * 32), vel_table.reshape(17 * 32))
